# Initial kernel scaffold; baseline (speedup 1.0000x reference)
import functools

import jax
import jax.numpy as jnp
from jax import lax
from jax.experimental import pallas as pl
from jax.experimental.pallas import tpu as pltpu
import jax.experimental.pallas.tpu_sc as plsc

N = 10000
E = 320000
NFEAT = 128
NHID = 64
ANO = 500

NC = 2
NS = 16
L = 16
NW = NC * NS

EPW = E // NW
CH = 80
NCHUNK = EPW // CH
ROWS_PER_TILE = N // NS
ZR = 25


def _make_spmm(F):
    mesh = plsc.VectorSubcoreMesh(core_axis_name="c", subcore_axis_name="s")

    @functools.partial(
        pl.kernel,
        out_type=jax.ShapeDtypeStruct((NC, N, F), jnp.float32),
        mesh=mesh,
        scratch_types=[
            pltpu.VMEM((CH,), jnp.int32),
            pltpu.VMEM((CH,), jnp.int32),
            pltpu.VMEM((CH, F), jnp.float32),
            pltpu.VMEM((ZR, F), jnp.float32),
            pltpu.VMEM_SHARED((N, F), jnp.float32),
            pltpu.SemaphoreType.DMA,
        ],
    )
    def spmm(support, src, dst, out, sidx, didx, rows, zbuf, acc, sem):
        c = lax.axis_index("c")
        s = lax.axis_index("s")
        wid = c * NS + s

        for r in range(ZR):
            for k in range(F // L):
                zbuf[r, pl.ds(k * L, L)] = jnp.zeros((L,), jnp.float32)
        r0 = s * ROWS_PER_TILE
        def zero_body(rep, _):
            pltpu.sync_copy(zbuf, acc.at[pl.ds(r0 + rep * ZR, ZR)])
            return 0
        lax.fori_loop(0, ROWS_PER_TILE // ZR, zero_body, 0)
        plsc.subcore_barrier()

        base = wid * EPW
        def body(j, _):
            off = base + j * CH
            pltpu.sync_copy(src.at[pl.ds(off, CH)], sidx)
            pltpu.sync_copy(dst.at[pl.ds(off, CH)], didx)
            pltpu.async_copy(support.at[sidx], rows, sem).wait()
            pltpu.sync_copy(rows, acc.at[didx], add=True)
            return 0
        lax.fori_loop(0, NCHUNK, body, 0)
        plsc.subcore_barrier()

        pltpu.sync_copy(acc.at[pl.ds(r0, ROWS_PER_TILE)],
                        out.at[c, pl.ds(r0, ROWS_PER_TILE)])

    return spmm


_spmm128 = _make_spmm(NFEAT)
_spmm64 = _make_spmm(NHID)


_BM = 1000


def _mm_body(x_ref, w_ref, o_ref):
    o_ref[...] = jnp.dot(x_ref[...], w_ref[...],
                         preferred_element_type=jnp.float32)


def _support1(nodes, W1):
    return pl.pallas_call(
        _mm_body,
        grid=(N // _BM,),
        in_specs=[
            pl.BlockSpec((_BM, NFEAT), lambda i: (i, 0)),
            pl.BlockSpec((NFEAT, 2 * NHID), lambda i: (0, 0)),
        ],
        out_specs=pl.BlockSpec((_BM, 2 * NHID), lambda i: (i, 0)),
        out_shape=jax.ShapeDtypeStruct((N, 2 * NHID), jnp.float32),
    )(nodes, W1)


def _layer2_body(p0_ref, p1_ref, b1_ref, w2_ref, o_ref):
    h = jax.nn.relu(p0_ref[...] + p1_ref[...] + b1_ref[...])
    o_ref[...] = jnp.dot(h, w2_ref[...], preferred_element_type=jnp.float32)


def _support2(p0, p1, b1, W2):
    F = 2 * NHID
    return pl.pallas_call(
        _layer2_body,
        grid=(N // _BM,),
        in_specs=[
            pl.BlockSpec((_BM, F), lambda i: (i, 0)),
            pl.BlockSpec((_BM, F), lambda i: (i, 0)),
            pl.BlockSpec((1, F), lambda i: (0, 0)),
            pl.BlockSpec((F, NHID), lambda i: (0, 0)),
        ],
        out_specs=pl.BlockSpec((_BM, NHID), lambda i: (i, 0)),
        out_shape=jax.ShapeDtypeStruct((N, NHID), jnp.float32),
    )(p0, p1, b1, W2)


def _score_body(p0_ref, p1_ref, b2_ref, wv_ref, bv_ref, o_ref):
    emb = p0_ref[...] + p1_ref[...] + b2_ref[...]
    logit = jnp.dot(emb, wv_ref[...],
                    preferred_element_type=jnp.float32) + bv_ref[...]
    o_ref[...] = jax.nn.sigmoid(logit)


def _scores(p0, p1, b2, Wv, bv):
    return pl.pallas_call(
        _score_body,
        grid=(N // _BM,),
        in_specs=[
            pl.BlockSpec((_BM, NHID), lambda i: (i, 0)),
            pl.BlockSpec((_BM, NHID), lambda i: (i, 0)),
            pl.BlockSpec((1, NHID), lambda i: (0, 0)),
            pl.BlockSpec((NHID, 1), lambda i: (0, 0)),
            pl.BlockSpec((1, 1), lambda i: (0, 0)),
        ],
        out_specs=pl.BlockSpec((_BM, 1), lambda i: (i, 0)),
        out_shape=jax.ShapeDtypeStruct((N, 1), jnp.float32),
    )(p0, p1, b2, Wv, bv)


def kernel(nodes, labels, adj, anomaly_list, norm_list, W1, b1, W2, b2, Wv, bv):
    src = adj[0]
    dst = adj[1]

    support1 = _support1(nodes, W1)
    part1 = _spmm128(support1, src, dst)
    support2 = _support2(part1[0], part1[1], b1.reshape(1, -1), W2)
    part2 = _spmm64(support2, src, dst)
    pred_score = _scores(part2[0], part2[1], b2.reshape(1, -1), Wv,
                         bv.reshape(1, 1))

    s = pred_score[:, 0]
    anomaly_scores = s[anomaly_list]
    norm_scores = s[norm_list]
    thresholds = 0.5 * (jnp.mean(anomaly_scores) + jnp.mean(norm_scores))
    topv, topi = jax.lax.top_k(s, ANO)
    mask_index = jnp.where(topv > thresholds, topi, -1)
    return (mask_index, thresholds, pred_score)


# speedup vs baseline: 4.5709x; 4.5709x over previous
import functools

import jax
import jax.numpy as jnp
from jax import lax
from jax.experimental import pallas as pl
from jax.experimental.pallas import tpu as pltpu
import jax.experimental.pallas.tpu_sc as plsc

N = 10000
E = 320000
NFEAT = 128
NHID = 64
ANO = 500

NC = 2
NS = 16
L = 16
NW = NC * NS

EPW = E // NW
CH = 80
NCHUNK = EPW // CH
RPT = 624
TAIL = N - NS * RPT
ZR = 48


def _make_spmm(F):
    mesh = plsc.VectorSubcoreMesh(core_axis_name="c", subcore_axis_name="s")

    @functools.partial(
        pl.kernel,
        out_type=jax.ShapeDtypeStruct((NC, N, F), jnp.float32),
        mesh=mesh,
        compiler_params=pltpu.CompilerParams(use_tc_tiling_on_sc=(F == 128)),
        scratch_types=[
            pltpu.VMEM((CH,), jnp.int32),
            pltpu.VMEM((CH,), jnp.int32),
            pltpu.VMEM((CH, F), jnp.float32),
            pltpu.VMEM((ZR, F), jnp.float32),
            pltpu.VMEM_SHARED((N, F), jnp.float32),
            pltpu.SemaphoreType.DMA,
        ],
    )
    def spmm(support, src, dst, out, sidx, didx, rows, zbuf, acc, sem):
        c = lax.axis_index("c")
        s = lax.axis_index("s")
        wid = c * NS + s

        def zfill(r, _):
            def zrow(k, _):
                zbuf[r, pl.ds(k * L, L)] = jnp.zeros((L,), jnp.float32)
                return 0
            lax.fori_loop(0, F // L, zrow, 0)
            return 0
        lax.fori_loop(0, ZR, zfill, 0)
        r0 = s * RPT
        def zero_body(rep, _):
            pltpu.sync_copy(zbuf, acc.at[pl.ds(r0 + rep * ZR, ZR)])
            return 0
        lax.fori_loop(0, RPT // ZR, zero_body, 0)

        @pl.when(s == NS - 1)
        def _zero_tail():
            pltpu.sync_copy(zbuf.at[pl.ds(0, TAIL)],
                            acc.at[pl.ds(NS * RPT, TAIL)])
        plsc.subcore_barrier()

        base = wid * EPW
        def body(j, _):
            off = base + j * CH
            pltpu.sync_copy(src.at[pl.ds(off, CH)], sidx)
            pltpu.sync_copy(dst.at[pl.ds(off, CH)], didx)
            pltpu.async_copy(support.at[sidx], rows, sem).wait()
            pltpu.sync_copy(rows, acc.at[didx], add=True)
            return 0
        lax.fori_loop(0, NCHUNK, body, 0)
        plsc.subcore_barrier()

        pltpu.sync_copy(acc.at[pl.ds(r0, RPT)], out.at[c, pl.ds(r0, RPT)])

        @pl.when(s == NS - 1)
        def _write_tail():
            pltpu.sync_copy(acc.at[pl.ds(NS * RPT, TAIL)],
                            out.at[c, pl.ds(NS * RPT, TAIL)])

    return spmm


_spmm128 = _make_spmm(NFEAT)
_spmm64 = _make_spmm(NHID)


_BM = 1000


def _mm_body(x_ref, w_ref, o_ref):
    o_ref[...] = jnp.dot(x_ref[...], w_ref[...],
                         preferred_element_type=jnp.float32)


def _support1(nodes, W1):
    return pl.pallas_call(
        _mm_body,
        grid=(N // _BM,),
        in_specs=[
            pl.BlockSpec((_BM, NFEAT), lambda i: (i, 0)),
            pl.BlockSpec((NFEAT, 2 * NHID), lambda i: (0, 0)),
        ],
        out_specs=pl.BlockSpec((_BM, 2 * NHID), lambda i: (i, 0)),
        out_shape=jax.ShapeDtypeStruct((N, 2 * NHID), jnp.float32),
    )(nodes, W1)


def _layer2_body(p0_ref, p1_ref, b1_ref, w2_ref, o_ref):
    h = jax.nn.relu(p0_ref[...] + p1_ref[...] + b1_ref[...])
    o_ref[...] = jnp.dot(h, w2_ref[...], preferred_element_type=jnp.float32)


def _support2(p0, p1, b1, W2):
    F = 2 * NHID
    return pl.pallas_call(
        _layer2_body,
        grid=(N // _BM,),
        in_specs=[
            pl.BlockSpec((_BM, F), lambda i: (i, 0)),
            pl.BlockSpec((_BM, F), lambda i: (i, 0)),
            pl.BlockSpec((1, F), lambda i: (0, 0)),
            pl.BlockSpec((F, NHID), lambda i: (0, 0)),
        ],
        out_specs=pl.BlockSpec((_BM, NHID), lambda i: (i, 0)),
        out_shape=jax.ShapeDtypeStruct((N, NHID), jnp.float32),
    )(p0, p1, b1, W2)


def _score_body(p0_ref, p1_ref, b2_ref, wv_ref, bv_ref, o_ref):
    emb = p0_ref[...] + p1_ref[...] + b2_ref[...]
    logit = jnp.dot(emb, wv_ref[...],
                    preferred_element_type=jnp.float32) + bv_ref[...]
    o_ref[...] = jax.nn.sigmoid(logit)


def _scores(p0, p1, b2, Wv, bv):
    return pl.pallas_call(
        _score_body,
        grid=(N // _BM,),
        in_specs=[
            pl.BlockSpec((_BM, NHID), lambda i: (i, 0)),
            pl.BlockSpec((_BM, NHID), lambda i: (i, 0)),
            pl.BlockSpec((1, NHID), lambda i: (0, 0)),
            pl.BlockSpec((NHID, 1), lambda i: (0, 0)),
            pl.BlockSpec((1, 1), lambda i: (0, 0)),
        ],
        out_specs=pl.BlockSpec((_BM, 1), lambda i: (i, 0)),
        out_shape=jax.ShapeDtypeStruct((N, 1), jnp.float32),
    )(p0, p1, b2, Wv, bv)


def kernel(nodes, labels, adj, anomaly_list, norm_list, W1, b1, W2, b2, Wv, bv):
    src = adj[0]
    dst = adj[1]

    support1 = _support1(nodes, W1)
    part1 = _spmm128(support1, src, dst)
    support2 = _support2(part1[0], part1[1], b1.reshape(1, -1), W2)
    part2 = _spmm64(support2, src, dst)
    pred_score = _scores(part2[0], part2[1], b2.reshape(1, -1), Wv,
                         bv.reshape(1, 1))

    s = pred_score[:, 0]
    anomaly_scores = s[anomaly_list]
    norm_scores = s[norm_list]
    thresholds = 0.5 * (jnp.mean(anomaly_scores) + jnp.mean(norm_scores))
    topv, topi = jax.lax.top_k(s, ANO)
    mask_index = jnp.where(topv > thresholds, topi, -1)
    return (mask_index, thresholds, pred_score)


# speedup vs baseline: 9.7200x; 2.1265x over previous
import functools

import jax
import jax.numpy as jnp
from jax import lax
from jax.experimental import pallas as pl
from jax.experimental.pallas import tpu as pltpu
import jax.experimental.pallas.tpu_sc as plsc

N = 10000
E = 320000
NFEAT = 128
NHID = 64
ANO = 500

NC = 2
NS = 16
L = 16
NW = NC * NS

EPW = E // NW
CH = 80
NCHUNK = EPW // CH
RPT = 624
TAIL = N - NS * RPT
ZR = 48


def _make_spmm(F):
    mesh = plsc.VectorSubcoreMesh(core_axis_name="c", subcore_axis_name="s")

    @functools.partial(
        pl.kernel,
        out_type=jax.ShapeDtypeStruct((NC, N, F), jnp.float32),
        mesh=mesh,
        compiler_params=pltpu.CompilerParams(use_tc_tiling_on_sc=(F == 128)),
        scratch_types=[
            pltpu.VMEM((EPW,), jnp.int32),
            pltpu.VMEM((EPW,), jnp.int32),
            pltpu.VMEM((CH,), jnp.int32),
            pltpu.VMEM((CH, F), jnp.float32),
            pltpu.VMEM((CH, F), jnp.float32),
            pltpu.VMEM((ZR, F), jnp.float32),
            pltpu.VMEM_SHARED((N, F), jnp.float32),
            pltpu.SemaphoreType.DMA,
            pltpu.SemaphoreType.DMA,
        ],
    )
    def spmm(support, src, dst, out, sidx_all, didx_all, didx, rows0, rows1,
             zbuf, acc, sem0, sem1):
        c = lax.axis_index("c")
        s = lax.axis_index("s")
        wid = c * NS + s

        def zfill(r, _):
            def zrow(k, _):
                zbuf[r, pl.ds(k * L, L)] = jnp.zeros((L,), jnp.float32)
                return 0
            lax.fori_loop(0, F // L, zrow, 0)
            return 0
        lax.fori_loop(0, ZR, zfill, 0)
        r0 = s * RPT
        def zero_body(rep, _):
            pltpu.sync_copy(zbuf, acc.at[pl.ds(r0 + rep * ZR, ZR)])
            return 0
        lax.fori_loop(0, RPT // ZR, zero_body, 0)

        @pl.when(s == NS - 1)
        def _zero_tail():
            pltpu.sync_copy(zbuf.at[pl.ds(0, TAIL)],
                            acc.at[pl.ds(NS * RPT, TAIL)])
        plsc.subcore_barrier()

        base = wid * EPW
        pltpu.sync_copy(src.at[pl.ds(base, EPW)], sidx_all)
        pltpu.sync_copy(dst.at[pl.ds(base, EPW)], didx_all)

        def fire(g, buf, sem):
            pltpu.async_copy(support.at[sidx_all.at[pl.ds(g * CH, CH)]],
                             buf, sem)

        def do_chunk(g, buf, sem, nbuf, nsem, fire_next):
            if fire_next:
                fire(g + 1, nbuf, nsem)
            pltpu.make_async_copy(support.at[pl.ds(0, CH)], buf, sem).wait()
            def cp(k, _):
                didx[pl.ds(k * L, L)] = didx_all[pl.ds(g * CH + k * L, L)]
                return 0
            lax.fori_loop(0, CH // L, cp, 0)
            pltpu.sync_copy(buf, acc.at[didx], add=True)

        fire(0, rows0, sem0)
        def pair(go, _):
            g0 = go * 2
            do_chunk(g0, rows0, sem0, rows1, sem1, True)
            do_chunk(g0 + 1, rows1, sem1, rows0, sem0, True)
            return 0
        lax.fori_loop(0, (NCHUNK - 1) // 2, pair, 0)
        do_chunk(NCHUNK - 1, rows0, sem0, rows1, sem1, False)
        plsc.subcore_barrier()

        pltpu.sync_copy(acc.at[pl.ds(r0, RPT)], out.at[c, pl.ds(r0, RPT)])

        @pl.when(s == NS - 1)
        def _write_tail():
            pltpu.sync_copy(acc.at[pl.ds(NS * RPT, TAIL)],
                            out.at[c, pl.ds(NS * RPT, TAIL)])

    return spmm


_spmm128 = _make_spmm(NFEAT)
_spmm64 = _make_spmm(NHID)


_BM = 1000


def _mm_body(x_ref, w_ref, o_ref):
    o_ref[...] = jnp.dot(x_ref[...], w_ref[...],
                         preferred_element_type=jnp.float32)


def _support1(nodes, W1):
    return pl.pallas_call(
        _mm_body,
        grid=(N // _BM,),
        in_specs=[
            pl.BlockSpec((_BM, NFEAT), lambda i: (i, 0)),
            pl.BlockSpec((NFEAT, 2 * NHID), lambda i: (0, 0)),
        ],
        out_specs=pl.BlockSpec((_BM, 2 * NHID), lambda i: (i, 0)),
        out_shape=jax.ShapeDtypeStruct((N, 2 * NHID), jnp.float32),
    )(nodes, W1)


def _layer2_body(p0_ref, p1_ref, b1_ref, w2_ref, o_ref):
    h = jax.nn.relu(p0_ref[...] + p1_ref[...] + b1_ref[...])
    o_ref[...] = jnp.dot(h, w2_ref[...], preferred_element_type=jnp.float32)


def _support2(p0, p1, b1, W2):
    F = 2 * NHID
    return pl.pallas_call(
        _layer2_body,
        grid=(N // _BM,),
        in_specs=[
            pl.BlockSpec((_BM, F), lambda i: (i, 0)),
            pl.BlockSpec((_BM, F), lambda i: (i, 0)),
            pl.BlockSpec((1, F), lambda i: (0, 0)),
            pl.BlockSpec((F, NHID), lambda i: (0, 0)),
        ],
        out_specs=pl.BlockSpec((_BM, NHID), lambda i: (i, 0)),
        out_shape=jax.ShapeDtypeStruct((N, NHID), jnp.float32),
    )(p0, p1, b1, W2)


def _score_body(p0_ref, p1_ref, b2_ref, wv_ref, bv_ref, o_ref):
    emb = p0_ref[...] + p1_ref[...] + b2_ref[...]
    logit = jnp.dot(emb, wv_ref[...],
                    preferred_element_type=jnp.float32) + bv_ref[...]
    o_ref[...] = jax.nn.sigmoid(logit)


def _scores(p0, p1, b2, Wv, bv):
    return pl.pallas_call(
        _score_body,
        grid=(N // _BM,),
        in_specs=[
            pl.BlockSpec((_BM, NHID), lambda i: (i, 0)),
            pl.BlockSpec((_BM, NHID), lambda i: (i, 0)),
            pl.BlockSpec((1, NHID), lambda i: (0, 0)),
            pl.BlockSpec((NHID, 1), lambda i: (0, 0)),
            pl.BlockSpec((1, 1), lambda i: (0, 0)),
        ],
        out_specs=pl.BlockSpec((_BM, 1), lambda i: (i, 0)),
        out_shape=jax.ShapeDtypeStruct((N, 1), jnp.float32),
    )(p0, p1, b2, Wv, bv)


def kernel(nodes, labels, adj, anomaly_list, norm_list, W1, b1, W2, b2, Wv, bv):
    src = adj[0]
    dst = adj[1]

    support1 = _support1(nodes, W1)
    part1 = _spmm128(support1, src, dst)
    support2 = _support2(part1[0], part1[1], b1.reshape(1, -1), W2)
    part2 = _spmm64(support2, src, dst)
    pred_score = _scores(part2[0], part2[1], b2.reshape(1, -1), Wv,
                         bv.reshape(1, 1))

    s = pred_score[:, 0]
    anomaly_scores = s[anomaly_list]
    norm_scores = s[norm_list]
    thresholds = 0.5 * (jnp.mean(anomaly_scores) + jnp.mean(norm_scores))
    topv, topi = jax.lax.top_k(s, ANO)
    mask_index = jnp.where(topv > thresholds, topi, -1)
    return (mask_index, thresholds, pred_score)
